# Initial kernel scaffold; baseline (speedup 1.0000x reference)
import jax
import jax.numpy as jnp
from jax.experimental import pallas as pl


def kernel(hidden_states, top_weights, top_experts, Wg, Wu, Wd):
    raise NotImplementedError("write your pallas kernel here")


# speedup vs baseline: 1.9647x; 1.9647x over previous
import jax
import jax.numpy as jnp
from jax.experimental import pallas as pl
from jax.experimental.pallas import tpu as pltpu


def _moe_dense_kernel(tw_ref, te_ref, x_ref, wg_ref, wu_ref, wd_ref, out_ref):
    e = pl.program_id(0)
    x = x_ref[...]
    gate = jax.nn.silu(jnp.dot(x, wg_ref[0], preferred_element_type=jnp.float32))
    up = jnp.dot(x, wu_ref[0], preferred_element_type=jnp.float32)
    y = jnp.dot(gate * up, wd_ref[0], preferred_element_type=jnp.float32)
    mask = (te_ref[...] == e).astype(jnp.float32)
    w_e = jnp.sum(tw_ref[...] * mask, axis=-1, keepdims=True)
    contrib = w_e * y

    @pl.when(e == 0)
    def _():
        out_ref[...] = contrib

    @pl.when(e > 0)
    def _():
        out_ref[...] += contrib


def kernel(hidden_states, top_weights, top_experts, Wg, Wu, Wd):
    B, S, H = hidden_states.shape
    T = B * S
    E, _, F = Wg.shape
    x = hidden_states.reshape(T, H)
    te = top_experts.astype(jnp.int32)

    out = pl.pallas_call(
        _moe_dense_kernel,
        grid=(E,),
        in_specs=[
            pl.BlockSpec((T, top_weights.shape[1]), lambda e: (0, 0)),
            pl.BlockSpec((T, te.shape[1]), lambda e: (0, 0)),
            pl.BlockSpec((T, H), lambda e: (0, 0)),
            pl.BlockSpec((1, H, F), lambda e: (e, 0, 0)),
            pl.BlockSpec((1, H, F), lambda e: (e, 0, 0)),
            pl.BlockSpec((1, F, H), lambda e: (e, 0, 0)),
        ],
        out_specs=pl.BlockSpec((T, H), lambda e: (0, 0)),
        out_shape=jax.ShapeDtypeStruct((T, H), jnp.float32),
    )(top_weights, te, x, Wg, Wu, Wd)
    return out.reshape(B, S, H)


# speedup vs baseline: 1.9961x; 1.0160x over previous
import jax
import jax.numpy as jnp
from jax.experimental import pallas as pl
from jax.experimental.pallas import tpu as pltpu


def _moe_dense_kernel(tw_ref, te_ref, x_ref, wg_ref, wu_ref, wd_ref, out_ref):
    e = pl.program_id(0)
    wg = wg_ref[0].astype(jnp.bfloat16)
    wu = wu_ref[0].astype(jnp.bfloat16)
    wd = wd_ref[0].astype(jnp.bfloat16)
    T = x_ref.shape[0]
    C = 512
    for c in range(T // C):
        sl = pl.ds(c * C, C)
        x = x_ref[sl, :].astype(jnp.bfloat16)
        gate = jax.nn.silu(jnp.dot(x, wg, preferred_element_type=jnp.float32))
        up = jnp.dot(x, wu, preferred_element_type=jnp.float32)
        h = (gate * up).astype(jnp.bfloat16)
        y = jnp.dot(h, wd, preferred_element_type=jnp.float32)
        mask = (te_ref[sl, :] == e).astype(jnp.float32)
        w_e = jnp.sum(tw_ref[sl, :] * mask, axis=-1, keepdims=True)
        contrib = w_e * y

        @pl.when(e == 0)
        def _():
            out_ref[sl, :] = contrib

        @pl.when(e > 0)
        def _():
            out_ref[sl, :] += contrib


def kernel(hidden_states, top_weights, top_experts, Wg, Wu, Wd):
    B, S, H = hidden_states.shape
    T = B * S
    E, _, F = Wg.shape
    x = hidden_states.reshape(T, H)
    te = top_experts.astype(jnp.int32)

    out = pl.pallas_call(
        _moe_dense_kernel,
        grid=(E,),
        in_specs=[
            pl.BlockSpec((T, top_weights.shape[1]), lambda e: (0, 0)),
            pl.BlockSpec((T, te.shape[1]), lambda e: (0, 0)),
            pl.BlockSpec((T, H), lambda e: (0, 0)),
            pl.BlockSpec((1, H, F), lambda e: (e, 0, 0)),
            pl.BlockSpec((1, H, F), lambda e: (e, 0, 0)),
            pl.BlockSpec((1, F, H), lambda e: (e, 0, 0)),
        ],
        out_specs=pl.BlockSpec((T, H), lambda e: (0, 0)),
        out_shape=jax.ShapeDtypeStruct((T, H), jnp.float32),
    )(top_weights, te, x, Wg, Wu, Wd)
    return out.reshape(B, S, H)
